# Initial kernel scaffold; baseline (speedup 1.0000x reference)
import jax
import jax.numpy as jnp
from jax.experimental import pallas as pl


def kernel(x, edge_index, W):
    raise NotImplementedError("write your pallas kernel here")


# speedup vs baseline: 5.4143x; 5.4143x over previous
import functools

import jax
import jax.numpy as jnp
from jax import lax
from jax.experimental import pallas as pl
from jax.experimental.pallas import tpu as pltpu
import jax.experimental.pallas.tpu_sc as plsc

N_SC_CORES = 2
N_SUBCORES = 16
EDGE_CHUNK = 80


def _matmul_body(x_ref, w_ref, out_ref):
    out_ref[...] = jnp.dot(x_ref[...], w_ref[...],
                           preferred_element_type=jnp.float32)


@functools.partial(jax.jit, static_argnames=("block_m",))
def _matmul(x, w, block_m=2000):
    n, d_in = x.shape
    d_out = w.shape[1]
    return pl.pallas_call(
        _matmul_body,
        grid=(n // block_m,),
        in_specs=[
            pl.BlockSpec((block_m, d_in), lambda i: (i, 0)),
            pl.BlockSpec((d_in, d_out), lambda i: (0, 0)),
        ],
        out_specs=pl.BlockSpec((block_m, d_out), lambda i: (i, 0)),
        out_shape=jax.ShapeDtypeStruct((n, d_out), jnp.float32),
    )(x, w)


def _combine_body(p_ref, out_ref):
    out_ref[...] = p_ref[0] + p_ref[1]


@functools.partial(jax.jit, static_argnames=("block_m",))
def _combine(partials, block_m=2000):
    _, n, d = partials.shape
    return pl.pallas_call(
        _combine_body,
        grid=(n // block_m,),
        in_specs=[pl.BlockSpec((2, block_m, d), lambda i: (0, i, 0))],
        out_specs=pl.BlockSpec((block_m, d), lambda i: (i, 0)),
        out_shape=jax.ShapeDtypeStruct((n, d), jnp.float32),
    )(partials)


def _make_scatter(n_nodes, n_edges, d):
    band = (n_nodes // N_SUBCORES) // 8 * 8
    tail = n_nodes - band * N_SUBCORES
    edges_per_tile = n_edges // (N_SC_CORES * N_SUBCORES)
    n_chunks = edges_per_tile // EDGE_CHUNK
    mesh = plsc.VectorSubcoreMesh(core_axis_name="c", subcore_axis_name="s")

    @functools.partial(
        pl.kernel,
        out_type=jax.ShapeDtypeStruct((N_SC_CORES, n_nodes, d), jnp.float32),
        mesh=mesh,
        scratch_types=[
            pltpu.VMEM((EDGE_CHUNK,), jnp.int32),
            pltpu.VMEM((EDGE_CHUNK,), jnp.int32),
            pltpu.VMEM((EDGE_CHUNK, d), jnp.float32),
            pltpu.VMEM_SHARED((n_nodes, d), jnp.float32),
            pltpu.SemaphoreType.DMA,
        ],
    )
    def scatter_kernel(h, src, dst, zeros, out, src_v, dst_v, rows_v, acc,
                       sem):
        c = lax.axis_index("c")
        s = lax.axis_index("s")
        pltpu.sync_copy(zeros, acc.at[pl.ds(s * band, band)])

        @pl.when(s == 0)
        def _zero_tail():
            pltpu.sync_copy(zeros.at[pl.ds(0, tail)],
                            acc.at[pl.ds(N_SUBCORES * band, tail)])

        plsc.subcore_barrier()

        tile_base = (c * N_SUBCORES + s) * edges_per_tile

        def step(i, _):
            base = tile_base + i * EDGE_CHUNK
            pltpu.sync_copy(src.at[pl.ds(base, EDGE_CHUNK)], src_v)
            pltpu.sync_copy(dst.at[pl.ds(base, EDGE_CHUNK)], dst_v)
            pltpu.async_copy(h.at[src_v], rows_v, sem).wait()
            pltpu.sync_copy(rows_v, acc.at[dst_v], add=True)
            return 0

        lax.fori_loop(0, n_chunks, step, 0)
        plsc.subcore_barrier()
        r0 = s * band
        pltpu.sync_copy(acc.at[pl.ds(r0, band)], out.at[c, pl.ds(r0, band)])

        @pl.when(s == 0)
        def _write_tail():
            pltpu.sync_copy(acc.at[pl.ds(N_SUBCORES * band, tail)],
                            out.at[c, pl.ds(N_SUBCORES * band, tail)])

    return scatter_kernel


def kernel(x, edge_index, W):
    n_nodes, d_in = x.shape
    d_out = W.shape[1]
    n_edges = edge_index.shape[1]

    ei = edge_index.astype(jnp.int32)
    src = ei[0]
    dst = ei[1]
    band = (n_nodes // N_SUBCORES) // 8 * 8
    zeros = jnp.zeros((band, d_out), jnp.float32)

    h = _matmul(x, W)
    scatter = _make_scatter(n_nodes, n_edges, d_out)
    partials = scatter(h, src, dst, zeros)
    return _combine(partials)


# speedup vs baseline: 8.3807x; 1.5479x over previous
import functools

import jax
import jax.numpy as jnp
from jax import lax
from jax.experimental import pallas as pl
from jax.experimental.pallas import tpu as pltpu
import jax.experimental.pallas.tpu_sc as plsc

N_SC_CORES = 2
N_SUBCORES = 16
EDGE_CHUNK = 80


def _matmul_body(x_ref, w_ref, out_ref):
    out_ref[...] = jnp.dot(x_ref[...], w_ref[...],
                           preferred_element_type=jnp.float32)


@functools.partial(jax.jit, static_argnames=("block_m",))
def _matmul(x, w, block_m=2000):
    n, d_in = x.shape
    d_out = w.shape[1]
    return pl.pallas_call(
        _matmul_body,
        grid=(n // block_m,),
        in_specs=[
            pl.BlockSpec((block_m, d_in), lambda i: (i, 0)),
            pl.BlockSpec((d_in, d_out), lambda i: (0, 0)),
        ],
        out_specs=pl.BlockSpec((block_m, d_out), lambda i: (i, 0)),
        out_shape=jax.ShapeDtypeStruct((n, d_out), jnp.float32),
    )(x, w)


def _combine_body(p_ref, out_ref):
    out_ref[...] = p_ref[0] + p_ref[1]


@functools.partial(jax.jit, static_argnames=("block_m",))
def _combine(partials, block_m=2000):
    _, n, d = partials.shape
    return pl.pallas_call(
        _combine_body,
        grid=(n // block_m,),
        in_specs=[pl.BlockSpec((2, block_m, d), lambda i: (0, i, 0))],
        out_specs=pl.BlockSpec((block_m, d), lambda i: (i, 0)),
        out_shape=jax.ShapeDtypeStruct((n, d), jnp.float32),
    )(partials)


def _make_scatter(n_nodes, n_edges, d):
    band = (n_nodes // N_SUBCORES) // 8 * 8
    tail = n_nodes - band * N_SUBCORES
    edges_per_tile = n_edges // (N_SC_CORES * N_SUBCORES)
    n_chunks = edges_per_tile // EDGE_CHUNK
    mesh = plsc.VectorSubcoreMesh(core_axis_name="c", subcore_axis_name="s")

    @functools.partial(
        pl.kernel,
        out_type=jax.ShapeDtypeStruct((N_SC_CORES, n_nodes, d), jnp.float32),
        mesh=mesh,
        scratch_types=[
            pltpu.VMEM((EDGE_CHUNK,), jnp.int32),
            pltpu.VMEM((EDGE_CHUNK,), jnp.int32),
            pltpu.VMEM((EDGE_CHUNK,), jnp.int32),
            pltpu.VMEM((EDGE_CHUNK,), jnp.int32),
            pltpu.VMEM((EDGE_CHUNK, d), jnp.float32),
            pltpu.VMEM((EDGE_CHUNK, d), jnp.float32),
            pltpu.VMEM_SHARED((n_nodes, d), jnp.float32),
            pltpu.SemaphoreType.DMA,
            pltpu.SemaphoreType.DMA,
        ],
    )
    def scatter_kernel(h, src, dst, zeros, out, src_v0, src_v1, dst_v0,
                       dst_v1, rows_v0, rows_v1, acc, g0, g1):
        c = lax.axis_index("c")
        s = lax.axis_index("s")
        pltpu.sync_copy(zeros, acc.at[pl.ds(s * band, band)])

        @pl.when(s == 0)
        def _zero_tail():
            pltpu.sync_copy(zeros.at[pl.ds(0, tail)],
                            acc.at[pl.ds(N_SUBCORES * band, tail)])

        plsc.subcore_barrier()

        tile_base = (c * N_SUBCORES + s) * edges_per_tile

        def idx_load(i, sv, dv):
            base = tile_base + i * EDGE_CHUNK
            pltpu.sync_copy(src.at[pl.ds(base, EDGE_CHUNK)], sv)
            pltpu.sync_copy(dst.at[pl.ds(base, EDGE_CHUNK)], dv)

        def gather_start(sv, rv, sem):
            pltpu.async_copy(h.at[sv], rv, sem)

        def gather_wait(sv, rv, sem):
            pltpu.make_async_copy(h.at[sv], rv, sem).wait()

        def scatter(rv, dv):
            pltpu.sync_copy(rv, acc.at[dv], add=True)

        idx_load(0, src_v0, dst_v0)
        gather_start(src_v0, rows_v0, g0)
        idx_load(1, src_v1, dst_v1)
        n_pairs = n_chunks // 2

        def pair(p, _):
            i0 = 2 * p
            gather_wait(src_v0, rows_v0, g0)
            gather_start(src_v1, rows_v1, g1)
            scatter(rows_v0, dst_v0)
            idx_load(i0 + 2, src_v0, dst_v0)
            gather_wait(src_v1, rows_v1, g1)
            gather_start(src_v0, rows_v0, g0)
            scatter(rows_v1, dst_v1)

            @pl.when(p < n_pairs - 1)
            def _prefetch_odd():
                idx_load(i0 + 3, src_v1, dst_v1)

            return 0

        lax.fori_loop(0, n_pairs, pair, 0)
        gather_wait(src_v0, rows_v0, g0)
        scatter(rows_v0, dst_v0)
        plsc.subcore_barrier()
        r0 = s * band
        pltpu.sync_copy(acc.at[pl.ds(r0, band)], out.at[c, pl.ds(r0, band)])

        @pl.when(s == 0)
        def _write_tail():
            pltpu.sync_copy(acc.at[pl.ds(N_SUBCORES * band, tail)],
                            out.at[c, pl.ds(N_SUBCORES * band, tail)])

    return scatter_kernel


def kernel(x, edge_index, W):
    n_nodes, d_in = x.shape
    d_out = W.shape[1]
    n_edges = edge_index.shape[1]

    ei = edge_index.astype(jnp.int32)
    src = ei[0]
    dst = ei[1]
    band = (n_nodes // N_SUBCORES) // 8 * 8
    zeros = jnp.zeros((band, d_out), jnp.float32)

    h = _matmul(x, W)
    scatter = _make_scatter(n_nodes, n_edges, d_out)
    partials = scatter(h, src, dst, zeros)
    return _combine(partials)


# speedup vs baseline: 9.4367x; 1.1260x over previous
import functools

import jax
import jax.numpy as jnp
from jax import lax
from jax.experimental import pallas as pl
from jax.experimental.pallas import tpu as pltpu
import jax.experimental.pallas.tpu_sc as plsc

N_SC_CORES = 2
N_SUBCORES = 16
EDGE_CHUNK = 80


def _matmul_body(x_ref, w_ref, out_ref):
    out_ref[...] = jnp.dot(x_ref[...], w_ref[...],
                           preferred_element_type=jnp.float32)


@functools.partial(jax.jit, static_argnames=("block_m",))
def _matmul(x, w, block_m=2000):
    n, d_in = x.shape
    d_out = w.shape[1]
    return pl.pallas_call(
        _matmul_body,
        grid=(n // block_m,),
        in_specs=[
            pl.BlockSpec((block_m, d_in), lambda i: (i, 0)),
            pl.BlockSpec((d_in, d_out), lambda i: (0, 0)),
        ],
        out_specs=pl.BlockSpec((block_m, d_out), lambda i: (i, 0)),
        out_shape=jax.ShapeDtypeStruct((n, d_out), jnp.float32),
    )(x, w)


def _combine_body(p_ref, out_ref):
    out_ref[...] = p_ref[0] + p_ref[1]


@functools.partial(jax.jit, static_argnames=("block_m",))
def _combine(partials, block_m=2000):
    _, n, d = partials.shape
    return pl.pallas_call(
        _combine_body,
        grid=(n // block_m,),
        in_specs=[pl.BlockSpec((2, block_m, d), lambda i: (0, i, 0))],
        out_specs=pl.BlockSpec((block_m, d), lambda i: (i, 0)),
        out_shape=jax.ShapeDtypeStruct((n, d), jnp.float32),
    )(partials)


def _make_scatter(n_nodes, n_edges, d):
    band = (n_nodes // N_SUBCORES) // 8 * 8
    tail = n_nodes - band * N_SUBCORES
    edges_per_tile = n_edges // (N_SC_CORES * N_SUBCORES)
    n_chunks = edges_per_tile // EDGE_CHUNK
    mesh = plsc.VectorSubcoreMesh(core_axis_name="c", subcore_axis_name="s")

    @functools.partial(
        pl.kernel,
        out_type=jax.ShapeDtypeStruct((N_SC_CORES, n_nodes, d), jnp.float32),
        mesh=mesh,
        scratch_types=[
            pltpu.VMEM((n_chunks, EDGE_CHUNK), jnp.int32),
            [pltpu.VMEM((EDGE_CHUNK,), jnp.int32) for _ in range(2)],
            [pltpu.VMEM((EDGE_CHUNK, d), jnp.float32) for _ in range(2)],
            pltpu.VMEM_SHARED((n_nodes, d), jnp.float32),
            [pltpu.SemaphoreType.DMA for _ in range(2)],
            [pltpu.SemaphoreType.DMA for _ in range(2)],
            [pltpu.SemaphoreType.DMA for _ in range(2)],
        ],
    )
    def scatter_kernel(h, src, dst, zeros, out, src_all, dv, rows, acc,
                       gsem, ssem, isem):
        c = lax.axis_index("c")
        s = lax.axis_index("s")
        pltpu.sync_copy(zeros, acc.at[pl.ds(s * band, band)])

        @pl.when(s == 0)
        def _zero_tail():
            pltpu.sync_copy(zeros.at[pl.ds(0, tail)],
                            acc.at[pl.ds(N_SUBCORES * band, tail)])

        tile = c * N_SUBCORES + s
        pltpu.sync_copy(src.at[tile], src_all)
        plsc.subcore_barrier()

        tile_base = tile * edges_per_tile

        def gather_start(i, b):
            pltpu.async_copy(h.at[src_all.at[i]], rows[b], gsem[b])

        def gather_wait(i, b):
            pltpu.make_async_copy(h.at[src_all.at[i]], rows[b],
                                  gsem[b]).wait()

        def dst_load_start(i, b):
            pltpu.async_copy(
                dst.at[pl.ds(tile_base + i * EDGE_CHUNK, EDGE_CHUNK)],
                dv[b], isem[b])

        def dst_load_wait(i, b):
            pltpu.make_async_copy(
                dst.at[pl.ds(tile_base + i * EDGE_CHUNK, EDGE_CHUNK)],
                dv[b], isem[b]).wait()

        def scatter_start(b):
            pltpu.async_copy(rows[b], acc.at[dv[b]], ssem[b], add=True)

        def scatter_wait(b):
            pltpu.make_async_copy(rows[b], acc.at[dv[b]], ssem[b]).wait()

        dst_load_start(0, 0)
        gather_start(0, 0)

        def pair(p, _):
            i0 = 2 * p

            @pl.when(p >= 1)
            def _drain_prev_scatter():
                scatter_wait(1)

            gather_wait(i0, 0)
            dst_load_wait(i0, 0)
            scatter_start(0)
            dst_load_start(i0 + 1, 1)
            gather_start(i0 + 1, 1)

            gather_wait(i0 + 1, 1)
            dst_load_wait(i0 + 1, 1)
            scatter_wait(0)
            scatter_start(1)
            dst_load_start(i0 + 2, 0)
            gather_start(i0 + 2, 0)
            return 0

        lax.fori_loop(0, n_chunks // 2, pair, 0)
        j = n_chunks - 1
        gather_wait(j, 0)
        dst_load_wait(j, 0)
        scatter_wait(1)
        scatter_start(0)
        scatter_wait(0)
        plsc.subcore_barrier()
        r0 = s * band
        pltpu.sync_copy(acc.at[pl.ds(r0, band)], out.at[c, pl.ds(r0, band)])

        @pl.when(s == 0)
        def _write_tail():
            pltpu.sync_copy(acc.at[pl.ds(N_SUBCORES * band, tail)],
                            out.at[c, pl.ds(N_SUBCORES * band, tail)])

    return scatter_kernel


def kernel(x, edge_index, W):
    n_nodes, d_in = x.shape
    d_out = W.shape[1]
    n_edges = edge_index.shape[1]

    ei = edge_index.astype(jnp.int32)
    n_tiles = N_SC_CORES * N_SUBCORES
    n_chunks = n_edges // (n_tiles * EDGE_CHUNK)
    src = ei[0].reshape(n_tiles, n_chunks, EDGE_CHUNK)
    dst = ei[1]
    band = (n_nodes // N_SUBCORES) // 8 * 8
    zeros = jnp.zeros((band, d_out), jnp.float32)

    h = _matmul(x, W)
    scatter = _make_scatter(n_nodes, n_edges, d_out)
    partials = scatter(h, src, dst, zeros)
    return _combine(partials)
